# baseline (speedup 1.0000x reference)
import functools

import jax
import jax.numpy as jnp
import numpy as np
from jax import lax
from jax.experimental import pallas as pl
from jax.experimental.pallas import tpu as pltpu
from jax.experimental.pallas import tpu_sc as plsc

N = 10000
E = 320000
HIDDEN = 128
NUM_HEADS = 8
HEAD_DIM = HIDDEN // NUM_HEADS
SCALING = HEAD_DIM ** (-0.5)

NC = 2
NS = 16
NW = NC * NS
BLK = 40
CHK = 8
NCHK = 32
EPW = NCHK * CHK * BLK
EPAD = NW * EPW
UNROLL = 4
NPAD = 10240
ROWS_PER_TILE = NPAD // NS

_PERM = np.arange(HIDDEN)
for _j in range(HIDDEN // 16):
    _PERM[16 * _j + 8:16 * _j + 16] = 16 * _j + 23 - np.arange(8, 16)
_PMAT = np.eye(HIDDEN, dtype=np.float32)[_PERM]

_ORDER = np.empty(2 * HIDDEN, dtype=np.int64)
for _p in range(2 * HIDDEN):
    _t, _r = divmod(_p, 32)
    _ORDER[_p] = 32 * _t + (_r // 2) + 16 * (_r % 2)
_KVPERM = np.concatenate([_PERM, _PERM + HIDDEN])[_ORDER]


def _proj_body(h_ref, wq_ref, wkv_ref, bq_ref, bkv_ref, q_ref, kv_ref):
    hb = h_ref[...]
    q_ref[...] = (jnp.dot(hb, wq_ref[...], preferred_element_type=jnp.float32)
                  + bq_ref[...]) * SCALING
    kv_ref[...] = (jnp.dot(hb, wkv_ref[...],
                           preferred_element_type=jnp.float32)
                   + bkv_ref[...])


def _project(h, wq_t, wkv_t, bq, bkv):
    rb = 1000
    grid = (N // rb,)
    return pl.pallas_call(
        _proj_body,
        grid=grid,
        in_specs=[pl.BlockSpec((rb, HIDDEN), lambda i: (i, 0)),
                  pl.BlockSpec((HIDDEN, HIDDEN), lambda i: (0, 0)),
                  pl.BlockSpec((HIDDEN, 2 * HIDDEN), lambda i: (0, 0)),
                  pl.BlockSpec((1, HIDDEN), lambda i: (0, 0)),
                  pl.BlockSpec((1, 2 * HIDDEN), lambda i: (0, 0))],
        out_specs=[pl.BlockSpec((rb, HIDDEN), lambda i: (i, 0)),
                   pl.BlockSpec((rb, 2 * HIDDEN), lambda i: (i, 0))],
        out_shape=[jax.ShapeDtypeStruct((N, HIDDEN), jnp.float32),
                   jax.ShapeDtypeStruct((N, 2 * HIDDEN), jnp.float32)],
    )(h, wq_t, wkv_t, bq.reshape(1, HIDDEN), bkv.reshape(1, 2 * HIDDEN))


def _edge_body(q_hbm, kv_hbm, idx_hbm, ew_hbm, zeros_hbm, out_hbm,
               ibr0, ibr1, ibc0, ibc1, ewb0, ewb1, qg0, qg1, kvg0, kvg1,
               msg0, msg1,
               acc_sh,
               sem_c0, sem_c1, sem_q0, sem_q1, sem_kv0, sem_kv1,
               sem_s0, sem_s1):
    c = lax.axis_index("c")
    s = lax.axis_index("s")
    wid = s * NC + c
    ibr = (ibr0, ibr1)
    ibc = (ibc0, ibc1)
    ewb = (ewb0, ewb1)
    qg = (qg0, qg1)
    kvg = (kvg0, kvg1)
    msg = (msg0, msg1)
    sem_c = (sem_c0, sem_c1)
    sem_q = (sem_q0, sem_q1)
    sem_kv = (sem_kv0, sem_kv1)
    sem_s = (sem_s0, sem_s1)

    c0 = pltpu.async_copy(idx_hbm.at[0, pl.ds(wid * (NCHK * CHK), CHK)],
                          ibr[0], sem_c[0])
    c0c = pltpu.async_copy(idx_hbm.at[1, pl.ds(wid * (NCHK * CHK), CHK)],
                           ibc[0], sem_c[0])
    c0b = pltpu.async_copy(
        ew_hbm.at[pl.ds(wid * EPW, CHK * BLK)], ewb[0], sem_c[0])
    pltpu.sync_copy(zeros_hbm.at[pl.ds(s * ROWS_PER_TILE, ROWS_PER_TILE)],
                    acc_sh.at[pl.ds(s * ROWS_PER_TILE, ROWS_PER_TILE)])
    plsc.subcore_barrier()
    c0.wait()
    c0c.wait()
    c0b.wait()

    pltpu.sync_copy(zeros_hbm.at[pl.ds(0, BLK)], msg[0])
    pltpu.sync_copy(zeros_hbm.at[pl.ds(0, BLK)], msg[1])
    pltpu.async_copy(msg[0], acc_sh.at[ibr[0].at[0]], sem_s[0], add=True)
    pltpu.async_copy(msg[1], acc_sh.at[ibr[0].at[0]], sem_s[1], add=True)

    def fire_ichunk(ci_next, cb_next):
        base = wid * (NCHK * CHK) + ci_next * CHK
        pltpu.async_copy(idx_hbm.at[0, pl.ds(base, CHK)], ibr[cb_next],
                         sem_c[cb_next])
        pltpu.async_copy(idx_hbm.at[1, pl.ds(base, CHK)], ibc[cb_next],
                         sem_c[cb_next])
        pltpu.async_copy(
            ew_hbm.at[pl.ds(wid * EPW + ci_next * (CHK * BLK), CHK * BLK)],
            ewb[cb_next], sem_c[cb_next])

    def wait_ichunk(ci_next, cb_next):
        base = wid * (NCHK * CHK) + ci_next * CHK
        pltpu.make_async_copy(idx_hbm.at[0, pl.ds(base, CHK)], ibr[cb_next],
                              sem_c[cb_next]).wait()
        pltpu.make_async_copy(idx_hbm.at[1, pl.ds(base, CHK)], ibc[cb_next],
                              sem_c[cb_next]).wait()
        pltpu.make_async_copy(
            ew_hbm.at[pl.ds(wid * EPW + ci_next * (CHK * BLK), CHK * BLK)],
            ewb[cb_next], sem_c[cb_next]).wait()

    def fire_gather(cb, lb, p):
        pltpu.async_copy(q_hbm.at[ibr[cb].at[lb]], qg[p], sem_q[p])
        pltpu.async_copy(kv_hbm.at[ibc[cb].at[lb]], kvg[p], sem_kv[p])

    def wait_gather(cb, lb, p):
        pltpu.make_async_copy(q_hbm.at[ibr[cb].at[lb]], qg[p],
                              sem_q[p]).wait()
        pltpu.make_async_copy(kv_hbm.at[ibc[cb].at[lb]], kvg[p],
                              sem_kv[p]).wait()

    def compute(cb, lb, p):
        qr, kvr = qg[p], kvg[p]
        pltpu.make_async_copy(msg[p], acc_sh.at[ibr[cb].at[lb]],
                              sem_s[p]).wait()

        @plsc.parallel_loop(0, BLK, 1, unroll=UNROLL)
        def edge_loop(e):
            qv = [qr[e, pl.ds(16 * j, 16)] for j in range(8)]
            kv_ = [kvr[e, pl.ds(16 * j, 16)] for j in range(8)]
            pr = [qv[j] * kv_[j] for j in range(8)]
            pr = [pr[2 * j] + pr[2 * j + 1] for j in range(4)]
            pr = [pr[2 * j] + pr[2 * j + 1] for j in range(2)]
            acc = pr[0] + pr[1]
            folded = acc + lax.rev(acc, (0,))
            ew_b = plsc.load_gather(
                ewb[cb], [jnp.full((16,), lb * BLK + e, jnp.int32)])
            sig = ew_b / (1.0 + jnp.exp(-folded))
            vv = [kvr[e, pl.ds(HIDDEN + 16 * j, 16)] for j in range(8)]
            ms = [sig * vv[j] for j in range(8)]
            for j in range(8):
                msg[p][e, pl.ds(16 * j, 16)] = ms[j]
        pltpu.async_copy(msg[p], acc_sh.at[ibr[cb].at[lb]], sem_s[p],
                         add=True)

    def chunk_body(ci, cb, first, last):
        if first:
            fire_gather(cb, 0, 0)

        def pair_body(j, carry):
            lb = 2 * j
            fire_gather(cb, lb + 1, 1)
            wait_gather(cb, lb, 0)
            compute(cb, lb, 0)
            fire_gather(cb, lb + 2, 0)
            wait_gather(cb, lb + 1, 1)
            compute(cb, lb + 1, 1)
            return carry

        lax.fori_loop(0, (CHK - 2) // 2, pair_body, 0)
        if not last:
            fire_ichunk(ci + 1, 1 - cb)
        fire_gather(cb, CHK - 1, 1)
        wait_gather(cb, CHK - 2, 0)
        compute(cb, CHK - 2, 0)
        if not last:
            wait_ichunk(ci + 1, 1 - cb)
            fire_gather(1 - cb, 0, 0)
        wait_gather(cb, CHK - 1, 1)
        compute(cb, CHK - 1, 1)

    chunk_body(0, 0, True, False)

    def chunk_pair(k, carry):
        ci = 2 * k + 1
        chunk_body(ci, 1, False, False)
        chunk_body(ci + 1, 0, False, False)
        return carry

    lax.fori_loop(0, (NCHK - 2) // 2, chunk_pair, 0)
    chunk_body(NCHK - 1, 1, False, True)

    pltpu.make_async_copy(msg[0], acc_sh.at[ibr[0].at[CHK - 2]],
                          sem_s[0]).wait()
    pltpu.make_async_copy(msg[1], acc_sh.at[ibr[0].at[CHK - 1]],
                          sem_s[1]).wait()

    plsc.subcore_barrier()
    pltpu.sync_copy(acc_sh.at[pl.ds(s * ROWS_PER_TILE, ROWS_PER_TILE)],
                    out_hbm.at[c, pl.ds(s * ROWS_PER_TILE, ROWS_PER_TILE)])


_edge_kernel = functools.partial(
    pl.kernel,
    out_type=jax.ShapeDtypeStruct((NC, NPAD, HIDDEN), jnp.float32),
    mesh=plsc.VectorSubcoreMesh(core_axis_name="c", subcore_axis_name="s"),
    compiler_params=pltpu.CompilerParams(needs_layout_passes=False),
    scratch_types=[
        pltpu.VMEM((CHK, BLK), jnp.int32),
        pltpu.VMEM((CHK, BLK), jnp.int32),
        pltpu.VMEM((CHK, BLK), jnp.int32),
        pltpu.VMEM((CHK, BLK), jnp.int32),
        pltpu.VMEM((CHK * BLK,), jnp.float32),
        pltpu.VMEM((CHK * BLK,), jnp.float32),
        pltpu.VMEM((BLK, HIDDEN), jnp.float32),
        pltpu.VMEM((BLK, HIDDEN), jnp.float32),
        pltpu.VMEM((BLK, 2 * HIDDEN), jnp.float32),
        pltpu.VMEM((BLK, 2 * HIDDEN), jnp.float32),
        pltpu.VMEM((BLK, HIDDEN), jnp.float32),
        pltpu.VMEM((BLK, HIDDEN), jnp.float32),
        pltpu.VMEM_SHARED((NPAD, HIDDEN), jnp.float32),
        pltpu.SemaphoreType.DMA,
        pltpu.SemaphoreType.DMA,
        pltpu.SemaphoreType.DMA,
        pltpu.SemaphoreType.DMA,
        pltpu.SemaphoreType.DMA,
        pltpu.SemaphoreType.DMA,
        pltpu.SemaphoreType.DMA,
        pltpu.SemaphoreType.DMA,
    ],
)(_edge_body)


def _add_body(p_ref, pm_ref, o_ref):
    o_ref[...] = jnp.dot(p_ref[0] + p_ref[1], pm_ref[...],
                         preferred_element_type=jnp.float32,
                         precision=lax.Precision.HIGHEST)


def _combine(partials):
    rb = 1024
    return pl.pallas_call(
        _add_body,
        grid=(NPAD // rb,),
        in_specs=[pl.BlockSpec((NC, rb, HIDDEN), lambda i: (0, i, 0)),
                  pl.BlockSpec((HIDDEN, HIDDEN), lambda i: (0, 0))],
        out_specs=pl.BlockSpec((rb, HIDDEN), lambda i: (i, 0)),
        out_shape=jax.ShapeDtypeStruct((N, HIDDEN), jnp.float32),
    )(partials, jnp.asarray(_PMAT))


def kernel(h, edge_index, edge_weight, Wq, bq, Wk, bk, Wv, bv):
    wkv_t = jnp.concatenate([Wk.T[:, _PERM], Wv.T[:, _PERM]], axis=1)
    bkv = jnp.concatenate([bk[_PERM], bv[_PERM]])
    q, kv = _project(h, Wq.T[:, _PERM], wkv_t, bq[_PERM], bkv)
    idx = jnp.pad(edge_index.astype(jnp.int32),
                  ((0, 0), (0, EPAD - E))).reshape(2, NW * NCHK * CHK, BLK)
    ew = jnp.pad(edge_weight.reshape(E), (0, EPAD - E))
    zeros = jnp.zeros((NPAD, HIDDEN), jnp.float32)
    partials = _edge_kernel(q, kv, idx, ew, zeros)
    return _combine(partials)


# speedup vs baseline: 1.9641x; 1.9641x over previous
import functools

import jax
import jax.numpy as jnp
import numpy as np
from jax import lax
from jax.experimental import pallas as pl
from jax.experimental.pallas import tpu as pltpu
from jax.experimental.pallas import tpu_sc as plsc

N = 10000
E = 320000
HIDDEN = 128
NUM_HEADS = 8
HEAD_DIM = HIDDEN // NUM_HEADS
SCALING = HEAD_DIM ** (-0.5)

NC = 2
NS = 16
NW = NC * NS
EPW = E // NW
BLK = 40
CHK = 10
NCHK = EPW // (BLK * CHK)
UNROLL = 4
NPAD = 10112
ROWS_PER_TILE = NPAD // NS

_PERM = np.arange(HIDDEN)
for _j in range(HIDDEN // 16):
    _PERM[16 * _j + 8:16 * _j + 16] = 16 * _j + 23 - np.arange(8, 16)
_PMAT = np.eye(HIDDEN, dtype=np.float32)[_PERM]

_ORDER = np.empty(2 * HIDDEN, dtype=np.int64)
for _p in range(2 * HIDDEN):
    _t, _r = divmod(_p, 32)
    _ORDER[_p] = 32 * _t + (_r // 2) + 16 * (_r % 2)
_KVPERM = np.concatenate([_PERM, _PERM + HIDDEN])[_ORDER]


def _proj_body(h_ref, wq_ref, wkv_ref, bq_ref, bkv_ref, q_ref, kv_ref):
    hb = h_ref[...]
    q_ref[...] = (jnp.dot(hb, wq_ref[...], preferred_element_type=jnp.float32)
                  + bq_ref[...]) * SCALING
    kv_ref[...] = (jnp.dot(hb, wkv_ref[...],
                           preferred_element_type=jnp.float32)
                   + bkv_ref[...])


def _project(h, wq_t, wkv_t, bq, bkv):
    rb = 1000
    grid = (N // rb,)
    return pl.pallas_call(
        _proj_body,
        grid=grid,
        in_specs=[pl.BlockSpec((rb, HIDDEN), lambda i: (i, 0)),
                  pl.BlockSpec((HIDDEN, HIDDEN), lambda i: (0, 0)),
                  pl.BlockSpec((HIDDEN, 2 * HIDDEN), lambda i: (0, 0)),
                  pl.BlockSpec((1, HIDDEN), lambda i: (0, 0)),
                  pl.BlockSpec((1, 2 * HIDDEN), lambda i: (0, 0))],
        out_specs=[pl.BlockSpec((rb, HIDDEN), lambda i: (i, 0)),
                   pl.BlockSpec((rb, 2 * HIDDEN), lambda i: (i, 0))],
        out_shape=[jax.ShapeDtypeStruct((N, HIDDEN), jnp.float32),
                   jax.ShapeDtypeStruct((N, 2 * HIDDEN), jnp.float32)],
    )(h, wq_t, wkv_t, bq.reshape(1, HIDDEN), bkv.reshape(1, 2 * HIDDEN))


def _edge_body(q_hbm, kv_hbm, idx_hbm, ew_hbm, zeros_hbm, out_hbm,
               ibr0, ibr1, ibc0, ibc1, ewb0, ewb1, qg0, qg1, kvg0, kvg1,
               msg0, msg1,
               acc_sh,
               sem_c0, sem_c1, sem_q0, sem_q1, sem_kv0, sem_kv1,
               sem_s0, sem_s1):
    c = lax.axis_index("c")
    s = lax.axis_index("s")
    wid = s * NC + c
    ibr = (ibr0, ibr1)
    ibc = (ibc0, ibc1)
    ewb = (ewb0, ewb1)
    qg = (qg0, qg1)
    kvg = (kvg0, kvg1)
    msg = (msg0, msg1)
    sem_c = (sem_c0, sem_c1)
    sem_q = (sem_q0, sem_q1)
    sem_kv = (sem_kv0, sem_kv1)
    sem_s = (sem_s0, sem_s1)

    c0 = pltpu.async_copy(idx_hbm.at[wid, 0, 0], ibr[0], sem_c[0])
    c0c = pltpu.async_copy(idx_hbm.at[wid, 0, 1], ibc[0], sem_c[0])
    c0b = pltpu.async_copy(
        ew_hbm.at[pl.ds(wid * EPW, CHK * BLK)], ewb[0], sem_c[0])
    pltpu.sync_copy(zeros_hbm.at[pl.ds(s * ROWS_PER_TILE, ROWS_PER_TILE)],
                    acc_sh.at[pl.ds(s * ROWS_PER_TILE, ROWS_PER_TILE)])
    plsc.subcore_barrier()
    c0.wait()
    c0c.wait()
    c0b.wait()

    pltpu.sync_copy(zeros_hbm.at[pl.ds(0, BLK)], msg[0])
    pltpu.sync_copy(zeros_hbm.at[pl.ds(0, BLK)], msg[1])
    pltpu.async_copy(msg[0], acc_sh.at[ibr[0].at[0]], sem_s[0], add=True)
    pltpu.async_copy(msg[1], acc_sh.at[ibr[0].at[0]], sem_s[1], add=True)

    def fire_ichunk(ci_next, cb_next):
        pltpu.async_copy(idx_hbm.at[wid, ci_next, 0], ibr[cb_next],
                         sem_c[cb_next])
        pltpu.async_copy(idx_hbm.at[wid, ci_next, 1], ibc[cb_next],
                         sem_c[cb_next])
        pltpu.async_copy(
            ew_hbm.at[pl.ds(wid * EPW + ci_next * (CHK * BLK), CHK * BLK)],
            ewb[cb_next], sem_c[cb_next])

    def wait_ichunk(ci_next, cb_next):
        pltpu.make_async_copy(idx_hbm.at[wid, ci_next, 0], ibr[cb_next],
                              sem_c[cb_next]).wait()
        pltpu.make_async_copy(idx_hbm.at[wid, ci_next, 1], ibc[cb_next],
                              sem_c[cb_next]).wait()
        pltpu.make_async_copy(
            ew_hbm.at[pl.ds(wid * EPW + ci_next * (CHK * BLK), CHK * BLK)],
            ewb[cb_next], sem_c[cb_next]).wait()

    def fire_gather(cb, lb, p):
        pltpu.async_copy(q_hbm.at[ibr[cb].at[lb]], qg[p], sem_q[p])
        pltpu.async_copy(kv_hbm.at[ibc[cb].at[lb]], kvg[p], sem_kv[p])

    def wait_gather(cb, lb, p):
        pltpu.make_async_copy(q_hbm.at[ibr[cb].at[lb]], qg[p],
                              sem_q[p]).wait()
        pltpu.make_async_copy(kv_hbm.at[ibc[cb].at[lb]], kvg[p],
                              sem_kv[p]).wait()

    def compute(cb, lb, p):
        qr, kvr = qg[p], kvg[p]
        pltpu.make_async_copy(msg[p], acc_sh.at[ibr[cb].at[lb]],
                              sem_s[p]).wait()

        @plsc.parallel_loop(0, BLK, 1, unroll=UNROLL)
        def edge_loop(e):
            qv = [qr[e, pl.ds(16 * j, 16)] for j in range(8)]
            kv_ = [kvr[e, pl.ds(16 * j, 16)] for j in range(8)]
            pr = [qv[j] * kv_[j] for j in range(8)]
            pr = [pr[2 * j] + pr[2 * j + 1] for j in range(4)]
            pr = [pr[2 * j] + pr[2 * j + 1] for j in range(2)]
            acc = pr[0] + pr[1]
            folded = acc + lax.rev(acc, (0,))
            ew_b = plsc.load_gather(
                ewb[cb], [jnp.full((16,), lb * BLK + e, jnp.int32)])
            sig = ew_b / (1.0 + jnp.exp(-folded))
            vv = [kvr[e, pl.ds(HIDDEN + 16 * j, 16)] for j in range(8)]
            ms = [sig * vv[j] for j in range(8)]
            for j in range(8):
                msg[p][e, pl.ds(16 * j, 16)] = ms[j]
        pltpu.async_copy(msg[p], acc_sh.at[ibr[cb].at[lb]], sem_s[p],
                         add=True)

    def chunk_body(ci, cb, first, last):
        if first:
            fire_gather(cb, 0, 0)

        def pair_body(j, carry):
            lb = 2 * j
            fire_gather(cb, lb + 1, 1)
            wait_gather(cb, lb, 0)
            compute(cb, lb, 0)
            fire_gather(cb, lb + 2, 0)
            wait_gather(cb, lb + 1, 1)
            compute(cb, lb + 1, 1)
            return carry

        lax.fori_loop(0, (CHK - 2) // 2, pair_body, 0)
        if not last:
            fire_ichunk(ci + 1, 1 - cb)
        fire_gather(cb, CHK - 1, 1)
        wait_gather(cb, CHK - 2, 0)
        compute(cb, CHK - 2, 0)
        if not last:
            wait_ichunk(ci + 1, 1 - cb)
            fire_gather(1 - cb, 0, 0)
        wait_gather(cb, CHK - 1, 1)
        compute(cb, CHK - 1, 1)

    chunk_body(0, 0, True, False)

    def chunk_pair(k, carry):
        ci = 2 * k + 1
        chunk_body(ci, 1, False, False)
        chunk_body(ci + 1, 0, False, False)
        return carry

    lax.fori_loop(0, (NCHK - 3) // 2, chunk_pair, 0)
    chunk_body(NCHK - 2, 1, False, False)
    chunk_body(NCHK - 1, 0, False, True)

    pltpu.make_async_copy(msg[0], acc_sh.at[ibr[0].at[CHK - 2]],
                          sem_s[0]).wait()
    pltpu.make_async_copy(msg[1], acc_sh.at[ibr[0].at[CHK - 1]],
                          sem_s[1]).wait()

    plsc.subcore_barrier()
    pltpu.sync_copy(acc_sh.at[pl.ds(s * ROWS_PER_TILE, ROWS_PER_TILE)],
                    out_hbm.at[c, pl.ds(s * ROWS_PER_TILE, ROWS_PER_TILE)])


_edge_kernel = functools.partial(
    pl.kernel,
    out_type=jax.ShapeDtypeStruct((NC, NPAD, HIDDEN), jnp.float32),
    mesh=plsc.VectorSubcoreMesh(core_axis_name="c", subcore_axis_name="s"),
    compiler_params=pltpu.CompilerParams(needs_layout_passes=False),
    scratch_types=[
        pltpu.VMEM((CHK, BLK), jnp.int32),
        pltpu.VMEM((CHK, BLK), jnp.int32),
        pltpu.VMEM((CHK, BLK), jnp.int32),
        pltpu.VMEM((CHK, BLK), jnp.int32),
        pltpu.VMEM((CHK * BLK,), jnp.float32),
        pltpu.VMEM((CHK * BLK,), jnp.float32),
        pltpu.VMEM((BLK, HIDDEN), jnp.float32),
        pltpu.VMEM((BLK, HIDDEN), jnp.float32),
        pltpu.VMEM((BLK, 2 * HIDDEN), jnp.float32),
        pltpu.VMEM((BLK, 2 * HIDDEN), jnp.float32),
        pltpu.VMEM((BLK, HIDDEN), jnp.float32),
        pltpu.VMEM((BLK, HIDDEN), jnp.float32),
        pltpu.VMEM_SHARED((NPAD, HIDDEN), jnp.float32),
        pltpu.SemaphoreType.DMA,
        pltpu.SemaphoreType.DMA,
        pltpu.SemaphoreType.DMA,
        pltpu.SemaphoreType.DMA,
        pltpu.SemaphoreType.DMA,
        pltpu.SemaphoreType.DMA,
        pltpu.SemaphoreType.DMA,
        pltpu.SemaphoreType.DMA,
    ],
)(_edge_body)


def _add_body(p_ref, pm_ref, o_ref):
    o_ref[...] = jnp.dot(p_ref[0] + p_ref[1], pm_ref[...],
                         preferred_element_type=jnp.float32,
                         precision=lax.Precision.HIGHEST)


def _combine(partials):
    rb = 1264
    return pl.pallas_call(
        _add_body,
        grid=(NPAD // rb,),
        in_specs=[pl.BlockSpec((NC, rb, HIDDEN), lambda i: (0, i, 0)),
                  pl.BlockSpec((HIDDEN, HIDDEN), lambda i: (0, 0))],
        out_specs=pl.BlockSpec((rb, HIDDEN), lambda i: (i, 0)),
        out_shape=jax.ShapeDtypeStruct((N, HIDDEN), jnp.float32),
    )(partials, jnp.asarray(_PMAT))


def kernel(h, edge_index, edge_weight, Wq, bq, Wk, bk, Wv, bv):
    wkv_t = jnp.concatenate([Wk.T[:, _PERM], Wv.T[:, _PERM]], axis=1)
    bkv = jnp.concatenate([bk[_PERM], bv[_PERM]])
    q, kv = _project(h, Wq.T[:, _PERM], wkv_t, bq[_PERM], bkv)
    row = edge_index[0].astype(jnp.int32).reshape(NW, NCHK, 1, CHK, BLK)
    col = edge_index[1].astype(jnp.int32).reshape(NW, NCHK, 1, CHK, BLK)
    idx = jnp.concatenate([row, col], axis=2)
    ew = edge_weight.reshape(E)
    zeros = jnp.zeros((NPAD, HIDDEN), jnp.float32)
    partials = _edge_kernel(q, kv, idx, ew, zeros)
    return _combine(partials)
